# baseline (speedup 1.0000x reference)
import functools

import jax
import jax.numpy as jnp
from jax import lax
from jax.experimental import pallas as pl
from jax.experimental.pallas import tpu as pltpu
from jax.experimental.pallas import tpu_sc as plsc

_NBUF = 4
_DIST = 2


def _gather_kernel(B0, B1, D, nw, r_per_w):
  mesh = plsc.VectorSubcoreMesh(core_axis_name="c", subcore_axis_name="s")
  n_chunks = r_per_w
  n_groups = n_chunks // _NBUF
  assert n_groups >= 2 and n_chunks % _NBUF == 0

  @functools.partial(
      pl.kernel,
      mesh=mesh,
      out_type=jax.ShapeDtypeStruct((B0, B1, D), jnp.float32),
      scratch_types=[
          pltpu.VMEM((r_per_w, B1), jnp.int32),
          [pltpu.VMEM((B1, D), jnp.float32)] * _NBUF,
          [pltpu.SemaphoreType.DMA] * _NBUF,
          [pltpu.SemaphoreType.DMA] * _NBUF,
      ],
      compiler_params=pltpu.CompilerParams(use_tc_tiling_on_sc=False),
  )
  def k(idx_hbm, table_hbm, out_hbm, idx_v, rows, gsems, wsems):
    nc = lax.axis_size("c")
    wid = lax.axis_index("s") * nc + lax.axis_index("c")
    base = wid * r_per_w
    pltpu.sync_copy(idx_hbm.at[pl.ds(base, r_per_w)], idx_v)

    def g_start(i, b):
      pltpu.async_copy(table_hbm.at[idx_v.at[i]], rows[b], gsems[b])

    def g_wait(i, b):
      pltpu.make_async_copy(table_hbm.at[idx_v.at[i]], rows[b], gsems[b]).wait()

    def w_start(i, b):
      pltpu.async_copy(rows[b], out_hbm.at[base + i], wsems[b])

    def w_wait(i, b):
      pltpu.make_async_copy(rows[b], out_hbm.at[base + i], wsems[b]).wait()

    for b in range(_DIST):
      g_start(b, b)

    for b in range(_NBUF):
      i = b
      g_wait(i, b)
      w_start(i, b)
      bj = (b + _DIST) % _NBUF
      if b >= _NBUF - _DIST:
        w_wait(i + _DIST - _NBUF, bj)
      g_start(i + _DIST, bj)

    @pl.loop(1, n_groups - 1)
    def _g(g):
      for b in range(_NBUF):
        i = g * _NBUF + b
        g_wait(i, b)
        w_start(i, b)
        bj = (b + _DIST) % _NBUF
        w_wait(i + _DIST - _NBUF, bj)
        g_start(i + _DIST, bj)

    tail = []
    for b in range(_NBUF):
      i = n_chunks - _NBUF + b
      g_wait(i, b)
      w_start(i, b)
      bj = (b + _DIST) % _NBUF
      w_wait(i + _DIST - _NBUF, bj)
      if b < _NBUF - _DIST:
        g_start(i + _DIST, bj)
      else:
        tail.append((i, b))
    for i, b in tail:
      w_wait(i, b)

  return k


def kernel(x, emb_weight):
  B0, B1 = x.shape
  V, D = emb_weight.shape
  nw = 32
  r_per_w = B0 // nw
  idx = x.astype(jnp.int32)
  return _gather_kernel(B0, B1, D, nw, r_per_w)(idx, emb_weight)


# speedup vs baseline: 1.3310x; 1.3310x over previous
import functools

import jax
import jax.numpy as jnp
from jax import lax
from jax.experimental import pallas as pl
from jax.experimental.pallas import tpu as pltpu
from jax.experimental.pallas import tpu_sc as plsc

_NBUF = 4
_DIST = 2


def _gather_kernel(B, D, n_per_w, chunk):
  mesh = plsc.VectorSubcoreMesh(core_axis_name="c", subcore_axis_name="s")
  n_chunks = n_per_w // chunk
  n_groups = n_chunks // _NBUF
  assert n_groups >= 2 and n_chunks % _NBUF == 0

  @functools.partial(
      pl.kernel,
      mesh=mesh,
      out_type=jax.ShapeDtypeStruct((B, 128), jnp.float32),
      scratch_types=[
          pltpu.VMEM((n_per_w,), jnp.int32),
          [pltpu.VMEM((chunk, D), jnp.float32)] * _NBUF,
          [pltpu.SemaphoreType.DMA] * _NBUF,
          [pltpu.SemaphoreType.DMA] * _NBUF,
      ],
      compiler_params=pltpu.CompilerParams(use_tc_tiling_on_sc=False),
  )
  def k(idx_hbm, table_hbm, out_hbm, idx_v, rows, gsems, wsems):
    nc = lax.axis_size("c")
    wid = lax.axis_index("s") * nc + lax.axis_index("c")
    base = wid * n_per_w
    pltpu.sync_copy(idx_hbm.at[pl.ds(base, n_per_w)], idx_v)

    def g_start(i, b):
      pltpu.async_copy(table_hbm.at[idx_v.at[pl.ds(i * chunk, chunk)]],
                       rows[b], gsems[b])

    def g_wait(i, b):
      pltpu.make_async_copy(table_hbm.at[idx_v.at[pl.ds(i * chunk, chunk)]],
                            rows[b], gsems[b]).wait()

    def w_start(i, b):
      pltpu.async_copy(rows[b],
                       out_hbm.at[pl.ds(base + i * chunk, chunk), pl.ds(0, D)],
                       wsems[b])

    def w_wait(i, b):
      pltpu.make_async_copy(
          rows[b],
          out_hbm.at[pl.ds(base + i * chunk, chunk), pl.ds(0, D)],
          wsems[b]).wait()

    for b in range(_DIST):
      g_start(b, b)

    for b in range(_NBUF):
      i = b
      g_wait(i, b)
      w_start(i, b)
      bj = (b + _DIST) % _NBUF
      if b >= _NBUF - _DIST:
        w_wait(i + _DIST - _NBUF, bj)
      g_start(i + _DIST, bj)

    @pl.loop(1, n_groups - 1)
    def _g(g):
      for b in range(_NBUF):
        i = g * _NBUF + b
        g_wait(i, b)
        w_start(i, b)
        bj = (b + _DIST) % _NBUF
        w_wait(i + _DIST - _NBUF, bj)
        g_start(i + _DIST, bj)

    tail = []
    for b in range(_NBUF):
      i = n_chunks - _NBUF + b
      g_wait(i, b)
      w_start(i, b)
      bj = (b + _DIST) % _NBUF
      w_wait(i + _DIST - _NBUF, bj)
      if b < _NBUF - _DIST:
        g_start(i + _DIST, bj)
      else:
        tail.append((i, b))
    for i, b in tail:
      w_wait(i, b)

  return k


def kernel(x, emb_weight):
  B0, B1 = x.shape
  V, D = emb_weight.shape
  B = B0 * B1
  nw = 32
  n_per_w = B // nw
  chunk = 200

  idx = x.reshape(B).astype(jnp.int32)
  out = _gather_kernel(B, D, n_per_w, chunk)(idx, emb_weight)
  return out[:, :D].reshape(B0, B1, D)
